# baseline (speedup 1.0000x reference)
import jax
import jax.numpy as jnp
from jax.experimental import pallas as pl


def _copy_body(x_ref, b_ref, xo_ref, bo_ref):
    xo_ref[...] = x_ref[...]
    bo_ref[...] = b_ref[...]


def kernel(x, edge_index, batch):
    del edge_index
    xo, bo = pl.pallas_call(
        _copy_body,
        out_shape=(
            jax.ShapeDtypeStruct(x.shape, x.dtype),
            jax.ShapeDtypeStruct(batch.shape, batch.dtype),
        ),
    )(x, batch)
    return (xo, bo)
